# baseline (speedup 1.0000x reference)
import functools

import jax
import jax.numpy as jnp
from jax import lax
from jax.experimental import pallas as pl
from jax.experimental.pallas import tpu as pltpu
from jax.experimental.pallas import tpu_sc as plsc

VOCAB = 100000
D = 128
B = 4096
L = 200

NC = 2
NS = 16
NW = NC * NS
CHUNK = 128
NQ = 4
QROWS = B // NQ
ITEM_CHUNKS = QROWS // CHUNK
QC = NQ * L
ITEMS = QC // NW
TOTAL = ITEMS * ITEM_CHUNKS
LANES = 16
NV = D // LANES
UNROLL = 8
NBUF = 5

VT = 8192
GRID_V = -(-VOCAB // VT)


def _accumulate(buf, acc):
    def body(j, acc):
        for r in range(UNROLL):
            row = j * UNROLL + r
            acc = tuple(
                acc[c] + buf[row, pl.ds(c * LANES, LANES)] for c in range(NV)
            )
        return acc

    return lax.fori_loop(0, CHUNK // UNROLL, body, acc)


def _pool_body(xT_hbm, table_hbm, out_hbm, idx_ref, buf0, buf1, buf2, buf3,
               buf4, res_ref, sem_idx, sem0, sem1, sem2, sem3, sem4):
    wid = lax.axis_index("s") * NC + lax.axis_index("c")
    bufs = (buf0, buf1, buf2, buf3, buf4)
    sems = (sem0, sem1, sem2, sem3, sem4)
    inv = jnp.float32(1.0 / B)

    def stage_body(i, _):
        pltpu.async_copy(
            xT_hbm.at[wid + i * NW],
            idx_ref.at[pl.ds(i * ITEM_CHUNKS, ITEM_CHUNKS)], sem_idx)
        return 0

    lax.fori_loop(0, ITEMS, stage_body, 0)
    for _ in range(ITEMS):
        pltpu.make_async_copy(
            xT_hbm.at[0], idx_ref.at[pl.ds(0, ITEM_CHUNKS)], sem_idx).wait()

    for s in range(NBUF):
        pltpu.async_copy(table_hbm.at[idx_ref.at[s]], bufs[s], sems[s])

    def group_body(g, acc):
        for s in range(NBUF):
            k = g * NBUF + s
            pltpu.make_async_copy(
                table_hbm.at[pl.ds(0, CHUNK)], bufs[s], sems[s]).wait()

            @pl.when(k + NBUF < TOTAL)
            def _():
                pltpu.async_copy(
                    table_hbm.at[idx_ref.at[k + NBUF]], bufs[s], sems[s])

            acc = _accumulate(bufs[s], acc)
            is_end = (k % ITEM_CHUNKS) == (ITEM_CHUNKS - 1)

            @pl.when(is_end)
            def _():
                item = k // ITEM_CHUNKS
                for c in range(NV):
                    res_ref[item, pl.ds(c * LANES, LANES)] = acc[c] * inv

            acc = tuple(
                jnp.where(is_end, jnp.zeros((LANES,), jnp.float32), a)
                for a in acc
            )
        return acc

    acc0 = tuple(jnp.zeros((LANES,), jnp.float32) for _ in range(NV))
    lax.fori_loop(0, TOTAL // NBUF, group_body, acc0)
    pltpu.sync_copy(res_ref, out_hbm.at[wid])


_pool = pl.kernel(
    _pool_body,
    out_type=jax.ShapeDtypeStruct((NW, ITEMS, D), jnp.float32),
    mesh=plsc.VectorSubcoreMesh(core_axis_name="c", subcore_axis_name="s"),
    scratch_types=[
        pltpu.VMEM((TOTAL, CHUNK), jnp.int32),
        pltpu.VMEM((CHUNK, D), jnp.float32),
        pltpu.VMEM((CHUNK, D), jnp.float32),
        pltpu.VMEM((CHUNK, D), jnp.float32),
        pltpu.VMEM((CHUNK, D), jnp.float32),
        pltpu.VMEM((CHUNK, D), jnp.float32),
        pltpu.VMEM((ITEMS, D), jnp.float32),
        pltpu.SemaphoreType.DMA,
        pltpu.SemaphoreType.DMA,
        pltpu.SemaphoreType.DMA,
        pltpu.SemaphoreType.DMA,
        pltpu.SemaphoreType.DMA,
        pltpu.SemaphoreType.DMA,
    ],
)


def _matmul_body(p_ref, w_ref, b_ref, o_ref):
    pooled = (p_ref[0] + p_ref[1]) + (p_ref[2] + p_ref[3])
    o_ref[...] = (
        lax.dot_general(
            pooled,
            w_ref[...],
            (((1,), (1,)), ((), ())),
            preferred_element_type=jnp.float32,
        )
        + b_ref[...]
    )


_matmul = pl.pallas_call(
    _matmul_body,
    grid=(GRID_V,),
    in_specs=[
        pl.BlockSpec((NQ, L, D), lambda i: (0, 0, 0)),
        pl.BlockSpec((VT, D), lambda i: (i, 0)),
        pl.BlockSpec((1, VT), lambda i: (0, i)),
    ],
    out_specs=pl.BlockSpec((L, VT), lambda i: (0, i)),
    out_shape=jax.ShapeDtypeStruct((L, VOCAB), jnp.float32),
)


def kernel(x, emb_table, W, b):
    xT = (
        x.T.astype(jnp.int32)
        .reshape(L, NQ, ITEM_CHUNKS, CHUNK)
        .reshape(QC, ITEM_CHUNKS, CHUNK)
    )
    flat = _pool(xT, emb_table)
    partials = (
        flat.transpose(1, 0, 2)
        .reshape(L, NQ, D)
        .transpose(1, 0, 2)
    )
    return _matmul(partials, W, b.reshape(1, VOCAB))


# speedup vs baseline: 1.0121x; 1.0121x over previous
import functools

import jax
import jax.numpy as jnp
from jax import lax
from jax.experimental import pallas as pl
from jax.experimental.pallas import tpu as pltpu
from jax.experimental.pallas import tpu_sc as plsc

VOCAB = 100000
D = 128
B = 4096
L = 200

NC = 2
NS = 16
NW = NC * NS
CHUNK = 128
NQ = 4
QROWS = B // NQ
ITEM_CHUNKS = QROWS // CHUNK
QC = NQ * L
ITEMS = QC // NW
TOTAL = ITEMS * ITEM_CHUNKS
LANES = 16
NV = D // LANES
UNROLL = 8
NBUF = 4

VT = 8192
GRID_V = -(-VOCAB // VT)


def _accumulate(buf, acc):
    def body(j, acc):
        for r in range(UNROLL):
            row = j * UNROLL + r
            acc = tuple(
                acc[c] + buf[row, pl.ds(c * LANES, LANES)] for c in range(NV)
            )
        return acc

    return lax.fori_loop(0, CHUNK // UNROLL, body, acc)


def _pool_body(xT_hbm, table_hbm, out_hbm, idx_ref, buf0, buf1, buf2, buf3,
               res_ref, sem_idx, sem0, sem1, sem2, sem3):
    wid = lax.axis_index("s") * NC + lax.axis_index("c")
    bufs = (buf0, buf1, buf2, buf3)
    sems = (sem0, sem1, sem2, sem3)
    inv = jnp.float32(1.0 / B)

    def stage_body(i, _):
        pltpu.async_copy(
            xT_hbm.at[wid + i * NW],
            idx_ref.at[pl.ds(i * ITEM_CHUNKS, ITEM_CHUNKS)], sem_idx)
        return 0

    lax.fori_loop(0, ITEMS, stage_body, 0)
    for _ in range(ITEMS):
        pltpu.make_async_copy(
            xT_hbm.at[0], idx_ref.at[pl.ds(0, ITEM_CHUNKS)], sem_idx).wait()

    for s in range(NBUF):
        pltpu.async_copy(table_hbm.at[idx_ref.at[s]], bufs[s], sems[s])

    def group_body(g, acc):
        for s in range(NBUF):
            k = g * NBUF + s
            pltpu.make_async_copy(
                table_hbm.at[pl.ds(0, CHUNK)], bufs[s], sems[s]).wait()

            @pl.when(k + NBUF < TOTAL)
            def _():
                pltpu.async_copy(
                    table_hbm.at[idx_ref.at[k + NBUF]], bufs[s], sems[s])

            acc = _accumulate(bufs[s], acc)
            is_end = (k % ITEM_CHUNKS) == (ITEM_CHUNKS - 1)

            @pl.when(is_end)
            def _():
                item = k // ITEM_CHUNKS
                for c in range(NV):
                    res_ref[item, pl.ds(c * LANES, LANES)] = acc[c] * inv

            acc = tuple(
                jnp.where(is_end, jnp.zeros((LANES,), jnp.float32), a)
                for a in acc
            )
        return acc

    acc0 = tuple(jnp.zeros((LANES,), jnp.float32) for _ in range(NV))
    lax.fori_loop(0, TOTAL // NBUF, group_body, acc0)
    pltpu.sync_copy(res_ref, out_hbm.at[wid])


_pool = pl.kernel(
    _pool_body,
    out_type=jax.ShapeDtypeStruct((NW, ITEMS, D), jnp.float32),
    mesh=plsc.VectorSubcoreMesh(core_axis_name="c", subcore_axis_name="s"),
    scratch_types=[
        pltpu.VMEM((TOTAL, CHUNK), jnp.int32),
        pltpu.VMEM((CHUNK, D), jnp.float32),
        pltpu.VMEM((CHUNK, D), jnp.float32),
        pltpu.VMEM((CHUNK, D), jnp.float32),
        pltpu.VMEM((CHUNK, D), jnp.float32),
        pltpu.VMEM((ITEMS, D), jnp.float32),
        pltpu.SemaphoreType.DMA,
        pltpu.SemaphoreType.DMA,
        pltpu.SemaphoreType.DMA,
        pltpu.SemaphoreType.DMA,
        pltpu.SemaphoreType.DMA,
    ],
)


def _matmul_body(p_ref, w_ref, b_ref, o_ref):
    pooled = (p_ref[0] + p_ref[1]) + (p_ref[2] + p_ref[3])
    o_ref[...] = (
        lax.dot_general(
            pooled,
            w_ref[...],
            (((1,), (1,)), ((), ())),
            preferred_element_type=jnp.float32,
        )
        + b_ref[...]
    )


_matmul = pl.pallas_call(
    _matmul_body,
    grid=(GRID_V,),
    in_specs=[
        pl.BlockSpec((NQ, L, D), lambda i: (0, 0, 0)),
        pl.BlockSpec((VT, D), lambda i: (i, 0)),
        pl.BlockSpec((1, VT), lambda i: (0, i)),
    ],
    out_specs=pl.BlockSpec((L, VT), lambda i: (0, i)),
    out_shape=jax.ShapeDtypeStruct((L, VOCAB), jnp.float32),
)


def kernel(x, emb_table, W, b):
    xT = (
        x.T.astype(jnp.int32)
        .reshape(L, NQ, ITEM_CHUNKS, CHUNK)
        .reshape(QC, ITEM_CHUNKS, CHUNK)
    )
    flat = _pool(xT, emb_table)
    partials = (
        flat.transpose(1, 0, 2)
        .reshape(L, NQ, D)
        .transpose(1, 0, 2)
    )
    return _matmul(partials, W, b.reshape(1, VOCAB))
